# baseline (speedup 1.0000x reference)
import functools

import jax
import jax.numpy as jnp
from jax import lax
from jax.experimental import pallas as pl
from jax.experimental.pallas import tpu as pltpu
from jax.experimental.pallas import tpu_sc as plsc

FIELDS = 26
DIM = 16
BATCH = 4096
VOCAB = 1000012
NC = 2
NW = 32
BPW = BATCH // NW
NTILE = 7813
NFULL = 7812
TAIL = VOCAB - NFULL * 128
PERW = 245


def _make_relayout_kernel():
    mesh = plsc.VectorSubcoreMesh(core_axis_name="c", subcore_axis_name="s")

    @functools.partial(
        pl.kernel,
        mesh=mesh,
        compiler_params=pltpu.CompilerParams(
            use_tc_tiling_on_sc=True, needs_layout_passes=False
        ),
        out_type=jax.ShapeDtypeStruct((VOCAB * DIM,), jnp.float32),
        scratch_types=[
            pltpu.VMEM((DIM, 128), jnp.float32),
            pltpu.VMEM((128 * DIM,), jnp.float32),
        ],
    )
    def relayout(embT_hbm, tail_hbm, flat_hbm, t_v, o_v):
        wid = lax.axis_index("s") * NC + lax.axis_index("c")
        lane16 = lax.iota(jnp.int32, 16)

        def step(j, carry):
            vc = wid * PERW + j

            @pl.when(vc < NFULL)
            def _():
                pltpu.sync_copy(embT_hbm.at[:, pl.ds(vc * 128, 128)], t_v)

                def col(c, carry2):
                    row = plsc.load_gather(
                        t_v, [lane16, jnp.full((16,), c, jnp.int32)]
                    )
                    o_v[pl.ds(c * DIM, DIM)] = row
                    return carry2

                lax.fori_loop(0, 128, col, 0)
                pltpu.sync_copy(
                    o_v, flat_hbm.at[pl.ds(vc * (128 * DIM), 128 * DIM)]
                )

            @pl.when(vc == NFULL)
            def _():
                pltpu.sync_copy(tail_hbm, t_v)

                def col(c, carry2):
                    row = plsc.load_gather(
                        t_v, [lane16, jnp.full((16,), c, jnp.int32)]
                    )
                    o_v[pl.ds(c * DIM, DIM)] = row
                    return carry2

                lax.fori_loop(0, TAIL, col, 0)
                pltpu.sync_copy(
                    o_v.at[pl.ds(0, TAIL * DIM)],
                    flat_hbm.at[pl.ds(NFULL * 128 * DIM, TAIL * DIM)],
                )

            return carry

        lax.fori_loop(0, PERW, step, 0)

    return relayout


def _make_gather_kernel():
    mesh = plsc.VectorSubcoreMesh(core_axis_name="c", subcore_axis_name="s")

    @functools.partial(
        pl.kernel,
        mesh=mesh,
        compiler_params=pltpu.CompilerParams(
            use_tc_tiling_on_sc=False, needs_layout_passes=False
        ),
        out_type=jax.ShapeDtypeStruct((FIELDS * DIM, NW, BPW), jnp.float32),
        scratch_types=[
            pltpu.VMEM((FIELDS, BPW), jnp.int32),
            pltpu.VMEM((FIELDS * BPW, DIM), jnp.float32),
            pltpu.VMEM((FIELDS * DIM, BPW), jnp.float32),
            pltpu.VMEM((FIELDS, DIM), jnp.float32),
            pltpu.SemaphoreType.DMA,
        ],
    )
    def gather(xT_hbm, emb_hbm, arch_hbm, out_hbm,
               idx_v, rows_v, tout_v, arch_v, sem):
        wid = lax.axis_index("s") * NC + lax.axis_index("c")
        b0 = wid * BPW

        pltpu.sync_copy(xT_hbm.at[:, pl.ds(b0, BPW)], idx_v)
        pltpu.sync_copy(arch_hbm, arch_v)

        def fire(f, carry):
            pltpu.async_copy(
                emb_hbm.at[idx_v.at[f]], rows_v.at[pl.ds(f * BPW, BPW)], sem
            )
            return carry

        lax.fori_loop(0, FIELDS, fire, 0)

        lane = lax.iota(jnp.int32, 16)

        pltpu.make_async_copy(out_hbm.at[:, wid], rows_v, sem).wait()

        def trans_f(f, carry):
            splat = arch_v[f]
            fd_idx = f * DIM + lane

            def trans_b(b, carry2):
                row = rows_v[f * BPW + b] * splat
                plsc.store_scatter(
                    tout_v, [fd_idx, jnp.full((16,), b, jnp.int32)], row
                )
                return carry2

            lax.fori_loop(0, BPW, trans_b, 0)
            return carry

        lax.fori_loop(0, FIELDS, trans_f, 0)
        pltpu.sync_copy(tout_v, out_hbm.at[:, wid])

    return gather


_relayout = _make_relayout_kernel()
_gather = _make_gather_kernel()


def kernel(x, embedding, arch):
    embT = embedding.T
    tail = jnp.pad(embedding[NFULL * 128:], ((0, 128 - TAIL), (0, 0))).T
    flat = _relayout(embT, tail)
    xT = x.T
    arch_b = jnp.broadcast_to(arch[:, None], (FIELDS, DIM))
    out = _gather(xT, flat.reshape(VOCAB, DIM), arch_b)
    return out.reshape(FIELDS, DIM, BATCH).transpose(2, 0, 1)


# speedup vs baseline: 1.4314x; 1.4314x over previous
import functools

import jax
import jax.numpy as jnp
from jax import lax
from jax.experimental import pallas as pl
from jax.experimental.pallas import tpu as pltpu
from jax.experimental.pallas import tpu_sc as plsc

FIELDS = 26
DIM = 16
BATCH = 4096
VOCAB = 1000012
NC = 2
NW = 32
BPW = BATCH // NW
NTILE = 7813
NFULL = 7812
TAIL = VOCAB - NFULL * 128
PERW = 245


def _make_relayout_kernel():
    mesh = plsc.VectorSubcoreMesh(core_axis_name="c", subcore_axis_name="s")

    @functools.partial(
        pl.kernel,
        mesh=mesh,
        compiler_params=pltpu.CompilerParams(
            use_tc_tiling_on_sc=True, needs_layout_passes=False
        ),
        out_type=jax.ShapeDtypeStruct((VOCAB * DIM,), jnp.float32),
        scratch_types=[
            pltpu.VMEM((DIM, 128), jnp.float32),
            pltpu.VMEM((DIM, 128), jnp.float32),
            pltpu.VMEM((128 * DIM,), jnp.float32),
            pltpu.VMEM((128 * DIM,), jnp.float32),
            pltpu.SemaphoreType.DMA,
            pltpu.SemaphoreType.DMA,
            pltpu.SemaphoreType.DMA,
            pltpu.SemaphoreType.DMA,
        ],
    )
    def relayout(embT_hbm, tail_hbm, flat_hbm, t0_v, t1_v, o0_v, o1_v,
                 rs0, rs1, ws0, ws1):
        wid = lax.axis_index("s") * NC + lax.axis_index("c")
        lane16 = lax.iota(jnp.int32, 16)
        vb = wid * PERW

        def fetch(vc, t_v, rs):
            @pl.when(vc < NFULL)
            def _():
                pltpu.async_copy(
                    embT_hbm.at[:, pl.ds(vc * 128, 128)], t_v, rs
                )

            @pl.when(vc == NFULL)
            def _():
                pltpu.async_copy(tail_hbm, t_v, rs)

        fetch(vb, t0_v, rs0)
        fetch(vb + 1, t1_v, rs1)

        def phase(k, j, t_v, o_v, rs, ws):
            vc = vb + j

            @pl.when((j < PERW) & (vc <= NFULL))
            def _():
                pltpu.make_async_copy(tail_hbm, t_v, rs).wait()

                @pl.when(k >= 1)
                def _():
                    pltpu.make_async_copy(
                        flat_hbm.at[pl.ds(0, 128 * DIM)], o_v, ws
                    ).wait()

                def col(c, carry2):
                    row = plsc.load_gather(
                        t_v, [lane16, jnp.full((16,), c, jnp.int32)]
                    )
                    o_v[pl.ds(c * DIM, DIM)] = row
                    return carry2

                lax.fori_loop(0, 128, col, 0)

                @pl.when(vc < NFULL)
                def _():
                    pltpu.async_copy(
                        o_v,
                        flat_hbm.at[pl.ds(vc * (128 * DIM), 128 * DIM)],
                        ws,
                    )

                @pl.when(vc == NFULL)
                def _():
                    pltpu.async_copy(
                        o_v.at[pl.ds(0, TAIL * DIM)],
                        flat_hbm.at[pl.ds(NFULL * 128 * DIM, TAIL * DIM)],
                        ws,
                    )

                fetch(vc + 2, t_v, rs)

        def step2(k, carry):
            phase(k, 2 * k, t0_v, o0_v, rs0, ws0)
            phase(k, 2 * k + 1, t1_v, o1_v, rs1, ws1)
            return carry

        lax.fori_loop(0, (PERW + 1) // 2, step2, 0)

        @pl.when(wid < NW - 1)
        def _():
            pltpu.make_async_copy(tail_hbm, t0_v, rs0).wait()
            pltpu.make_async_copy(tail_hbm, t1_v, rs1).wait()

        pltpu.make_async_copy(
            flat_hbm.at[pl.ds(0, 128 * DIM)], o0_v, ws0
        ).wait()

        @pl.when(wid < NW - 1)
        def _():
            pltpu.make_async_copy(
                flat_hbm.at[pl.ds(0, 128 * DIM)], o1_v, ws1
            ).wait()

        @pl.when(wid == NW - 1)
        def _():
            pltpu.make_async_copy(
                flat_hbm.at[pl.ds(0, TAIL * DIM)],
                o1_v.at[pl.ds(0, TAIL * DIM)],
                ws1,
            ).wait()

    return relayout


def _make_gather_kernel():
    mesh = plsc.VectorSubcoreMesh(core_axis_name="c", subcore_axis_name="s")

    @functools.partial(
        pl.kernel,
        mesh=mesh,
        compiler_params=pltpu.CompilerParams(
            use_tc_tiling_on_sc=False, needs_layout_passes=False
        ),
        out_type=jax.ShapeDtypeStruct((FIELDS * DIM, NW, BPW), jnp.float32),
        scratch_types=[
            pltpu.VMEM((FIELDS, BPW), jnp.int32),
            pltpu.VMEM((FIELDS * BPW, DIM), jnp.float32),
            pltpu.VMEM((FIELDS * DIM, BPW), jnp.float32),
            pltpu.VMEM((FIELDS, DIM), jnp.float32),
            pltpu.SemaphoreType.DMA,
        ],
    )
    def gather(xT_hbm, emb_hbm, arch_hbm, out_hbm,
               idx_v, rows_v, tout_v, arch_v, sem):
        wid = lax.axis_index("s") * NC + lax.axis_index("c")
        b0 = wid * BPW

        pltpu.sync_copy(xT_hbm.at[:, pl.ds(b0, BPW)], idx_v)
        pltpu.sync_copy(arch_hbm, arch_v)

        def fire(f, carry):
            pltpu.async_copy(
                emb_hbm.at[idx_v.at[f]], rows_v.at[pl.ds(f * BPW, BPW)], sem
            )
            return carry

        lax.fori_loop(0, FIELDS, fire, 0)

        lane = lax.iota(jnp.int32, 16)

        pltpu.make_async_copy(out_hbm.at[:, wid], rows_v, sem).wait()

        def trans_f(f, carry):
            splat = arch_v[f]
            fd_idx = f * DIM + lane

            def trans_b(b4, carry2):
                b = b4 * 4
                for u in range(4):
                    row = rows_v[f * BPW + b + u] * splat
                    plsc.store_scatter(
                        tout_v,
                        [fd_idx, jnp.full((16,), b + u, jnp.int32)],
                        row,
                    )
                return carry2

            lax.fori_loop(0, BPW // 4, trans_b, 0)
            return carry

        lax.fori_loop(0, FIELDS, trans_f, 0)
        pltpu.sync_copy(tout_v, out_hbm.at[:, wid])

    return gather


_relayout = _make_relayout_kernel()
_gather = _make_gather_kernel()


def kernel(x, embedding, arch):
    embT = embedding.T
    tail = jnp.pad(embedding[NFULL * 128:], ((0, 128 - TAIL), (0, 0))).T
    flat = _relayout(embT, tail)
    xT = x.T
    arch_b = jnp.broadcast_to(arch[:, None], (FIELDS, DIM))
    out = _gather(xT, flat.reshape(VOCAB, DIM), arch_b)
    return out.reshape(FIELDS, DIM, BATCH).transpose(2, 0, 1)
